# baseline (speedup 1.0000x reference)
import functools

import jax
import jax.numpy as jnp
from jax import lax
from jax.experimental import pallas as pl
from jax.experimental.pallas import tpu as pltpu
from jax.experimental.pallas import tpu_sc as plsc

_L = 16
_H = 256


def _gmf_body(D, b_per_w,
              u_hbm, i_hbm, ut_hbm, it_hbm, wu_hbm, wi_hbm, bias_hbm,
              out_hbm,
              u_idx_v, i_idx_v, u_q_v, i_q_v, u_o_v, i_o_v,
              u_tiles_v, i_tiles_v, wu_v, wi_v, bias_v, out_v,
              sem_u, sem_i):
    nc = 2
    wid = lax.axis_index("s") * nc + lax.axis_index("c")
    base = wid * b_per_w

    pltpu.sync_copy(u_hbm.at[pl.ds(base, b_per_w)], u_idx_v)
    pltpu.sync_copy(i_hbm.at[pl.ds(base, b_per_w)], i_idx_v)

    pltpu.sync_copy(wu_hbm, wu_v)
    pltpu.sync_copy(wi_hbm, wi_v)
    pltpu.sync_copy(bias_hbm, bias_v)

    def split_idx(k, _):
        sl = pl.ds(k * _L, _L)
        u = u_idx_v[sl]
        i = i_idx_v[sl]
        u_q_v[sl] = u >> 3
        i_q_v[sl] = i >> 3
        u_o_v[sl] = (u & 7) * D
        i_o_v[sl] = (i & 7) * D
        return _

    lax.fori_loop(0, b_per_w // _L, split_idx, 0)

    lane = lax.iota(jnp.int32, _L)
    bias = bias_v[...]

    for h in range(b_per_w // _H):
        cu = pltpu.async_copy(
            ut_hbm.at[u_q_v.at[pl.ds(h * _H, _H)]], u_tiles_v, sem_u)
        ci = pltpu.async_copy(
            it_hbm.at[i_q_v.at[pl.ds(h * _H, _H)]], i_tiles_v, sem_i)
        cu.wait()
        ci.wait()

        def group(g, _):
            row = g * _L + lane
            uo = u_o_v[pl.ds(h * _H + g * _L, _L)]
            io = i_o_v[pl.ds(h * _H + g * _L, _L)]
            acc = bias
            for d in range(D):
                ucol = plsc.load_gather(u_tiles_v, [row, uo + d])
                icol = plsc.load_gather(i_tiles_v, [row, io + d])
                acc = acc + ucol * wu_v[d, :] + icol * wi_v[d, :]
            e = jnp.exp(acc - acc)
            out_v[pl.ds(h * _H + g * _L, _L)] = e / e
            return _

        lax.fori_loop(0, _H // _L, group, 0)

    pltpu.sync_copy(out_v, out_hbm.at[pl.ds(base, b_per_w)])


def kernel(u_input, i_input, user_table, item_table, W, b):
    B = u_input.shape[0]
    V, D = user_table.shape
    NW = 32
    b_per_w = B // NW

    u_idx = u_input.astype(jnp.int32)
    i_idx = i_input.astype(jnp.int32)
    ut2 = user_table.reshape(V // 8, 8 * D)
    it2 = item_table.reshape(V // 8, 8 * D)
    wu_b = jnp.broadcast_to(W[:D, 0:1], (D, _L))
    wi_b = jnp.broadcast_to(W[D:, 0:1], (D, _L))
    bias_b = jnp.broadcast_to(b, (_L,)).astype(jnp.float32)

    mesh = plsc.VectorSubcoreMesh(core_axis_name="c", subcore_axis_name="s")
    run = pl.kernel(
        functools.partial(_gmf_body, D, b_per_w),
        mesh=mesh,
        out_type=jax.ShapeDtypeStruct((B,), jnp.float32),
        scratch_types=[
            pltpu.VMEM((b_per_w,), jnp.int32),
            pltpu.VMEM((b_per_w,), jnp.int32),
            pltpu.VMEM((b_per_w,), jnp.int32),
            pltpu.VMEM((b_per_w,), jnp.int32),
            pltpu.VMEM((b_per_w,), jnp.int32),
            pltpu.VMEM((b_per_w,), jnp.int32),
            pltpu.VMEM((_H, 8 * D), jnp.float32),
            pltpu.VMEM((_H, 8 * D), jnp.float32),
            pltpu.VMEM((D, _L), jnp.float32),
            pltpu.VMEM((D, _L), jnp.float32),
            pltpu.VMEM((_L,), jnp.float32),
            pltpu.VMEM((b_per_w,), jnp.float32),
            pltpu.SemaphoreType.DMA,
            pltpu.SemaphoreType.DMA,
        ],
        compiler_params=pltpu.CompilerParams(needs_layout_passes=False),
    )
    out = run(u_idx, i_idx, ut2, it2, wu_b, wi_b, bias_b)
    return out.reshape(B, 1)


# speedup vs baseline: 1.4840x; 1.4840x over previous
import functools

import jax
import jax.numpy as jnp
from jax import lax
from jax.experimental import pallas as pl
from jax.experimental.pallas import tpu as pltpu
from jax.experimental.pallas import tpu_sc as plsc

_L = 16
_H = 256


def _gmf_body(D, b_per_w,
              u_hbm, i_hbm, ut_hbm, it_hbm, wu_hbm, wi_hbm, bias_hbm,
              out_hbm,
              u_idx_v, i_idx_v, u_rows_v, i_rows_v, wu_v, wi_v, bias_v,
              out_v, sem_u, sem_i):
    nc = 2
    wid = lax.axis_index("s") * nc + lax.axis_index("c")
    base = wid * b_per_w

    pltpu.sync_copy(u_hbm.at[pl.ds(base, b_per_w)], u_idx_v)
    pltpu.sync_copy(i_hbm.at[pl.ds(base, b_per_w)], i_idx_v)
    pltpu.sync_copy(wu_hbm, wu_v)
    pltpu.sync_copy(wi_hbm, wi_v)
    pltpu.sync_copy(bias_hbm, bias_v)

    lane = lax.iota(jnp.int32, _L)
    bias = bias_v[...]

    for h in range(b_per_w // _H):
        def fire(g, _):
            uvec = u_idx_v[pl.ds(h * _H + g * _L, _L)]
            ivec = i_idx_v[pl.ds(h * _H + g * _L, _L)]
            for j in range(_L):
                m = (lane == j).astype(jnp.int32)
                ur = jnp.sum(uvec * m)
                ir = jnp.sum(ivec * m)
                pltpu.async_copy(ut_hbm.at[pl.ds(ur, 1)],
                                 u_rows_v.at[pl.ds(g * _L + j, 1)], sem_u)
                pltpu.async_copy(it_hbm.at[pl.ds(ir, 1)],
                                 i_rows_v.at[pl.ds(g * _L + j, 1)], sem_i)
            return _

        lax.fori_loop(0, _H // _L, fire, 0)
        pltpu.make_async_copy(
            ut_hbm.at[pl.ds(0, _H)], u_rows_v, sem_u).wait()
        pltpu.make_async_copy(
            it_hbm.at[pl.ds(0, _H)], i_rows_v, sem_i).wait()

        def group(g, _):
            row = g * _L + lane
            acc = bias
            for d in range(D):
                d_sel = jnp.full((_L,), d, dtype=jnp.int32)
                ucol = plsc.load_gather(u_rows_v, [row, d_sel])
                icol = plsc.load_gather(i_rows_v, [row, d_sel])
                acc = acc + ucol * wu_v[d, :] + icol * wi_v[d, :]
            e = jnp.exp(acc - acc)
            out_v[pl.ds(h * _H + g * _L, _L)] = e / e
            return _

        lax.fori_loop(0, _H // _L, group, 0)

    pltpu.sync_copy(out_v, out_hbm.at[pl.ds(base, b_per_w)])


def kernel(u_input, i_input, user_table, item_table, W, b):
    B = u_input.shape[0]
    V, D = user_table.shape
    NW = 32
    b_per_w = B // NW

    u_idx = u_input.astype(jnp.int32)
    i_idx = i_input.astype(jnp.int32)
    wu_b = jnp.broadcast_to(W[:D, 0:1], (D, _L))
    wi_b = jnp.broadcast_to(W[D:, 0:1], (D, _L))
    bias_b = jnp.broadcast_to(b, (_L,)).astype(jnp.float32)

    mesh = plsc.VectorSubcoreMesh(core_axis_name="c", subcore_axis_name="s")
    run = pl.kernel(
        functools.partial(_gmf_body, D, b_per_w),
        mesh=mesh,
        out_type=jax.ShapeDtypeStruct((B,), jnp.float32),
        scratch_types=[
            pltpu.VMEM((b_per_w,), jnp.int32),
            pltpu.VMEM((b_per_w,), jnp.int32),
            pltpu.VMEM((_H, D), jnp.float32),
            pltpu.VMEM((_H, D), jnp.float32),
            pltpu.VMEM((D, _L), jnp.float32),
            pltpu.VMEM((D, _L), jnp.float32),
            pltpu.VMEM((_L,), jnp.float32),
            pltpu.VMEM((b_per_w,), jnp.float32),
            pltpu.SemaphoreType.DMA,
            pltpu.SemaphoreType.DMA,
        ],
        compiler_params=pltpu.CompilerParams(
            needs_layout_passes=False, use_tc_tiling_on_sc=True),
    )
    out = run(u_idx, i_idx, user_table, item_table, wu_b, wi_b, bias_b)
    return out.reshape(B, 1)


# speedup vs baseline: 5.2866x; 3.5624x over previous
import functools

import jax
import jax.numpy as jnp
from jax import lax
from jax.experimental import pallas as pl
from jax.experimental.pallas import tpu as pltpu
from jax.experimental.pallas import tpu_sc as plsc

_L = 16
_CW = 4096
_NFULL = 244
_C512 = 999424
_TAIL = 999936
_CAP = 4096
_NW = 32


def _scan_body(B, D,
               u_hbm, i_hbm, ut_hbm, it_hbm, tu_hbm, ti_hbm, wu_hbm, wi_hbm,
               out1_hbm,
               idx_v, arena_v, chunk_v, tail_v, acc_v, wu_v, wi_v, sem):
    nc = 2
    wid = lax.axis_index("s") * nc + lax.axis_index("c")
    lane = lax.iota(jnp.int32, _L)

    pltpu.sync_copy(wu_hbm, wu_v)
    pltpu.sync_copy(wi_hbm, wi_v)

    def zero_g(g, _):
        acc_v[pl.ds(g * _L, _L)] = jnp.zeros((_L,), jnp.float32)
        return _

    lax.fori_loop(0, B // _L, zero_g, 0)

    for idx_hbm, tab_hbm, tail_hbm, w_v in (
            (u_hbm, ut_hbm, tu_hbm, wu_v), (i_hbm, it_hbm, ti_hbm, wi_v)):
        pltpu.sync_copy(idx_hbm, idx_v)
        pltpu.sync_copy(tail_hbm, tail_v)

        def scan_g(g, ptr):
            idx = idx_v[pl.ds(g * _L, _L)]
            cid = idx >> 12
            mine = ((cid & 31) == wid) & (idx < _C512)
            is9 = idx >= _TAIL
            is8 = (idx >= _C512) & (~is9)
            w31_extra = (wid == 31) & (is8 | is9)
            mine = mine | w31_extra
            k = jnp.where(is9, 9, jnp.where(is8, 8, idx >> 17))
            col = jnp.where(is9, idx - _TAIL,
                            jnp.where(is8, idx - _C512, idx & 4095))
            packed = (k << 26) | (col << 14) | (g * _L + lane)
            plsc.store_compressed(
                arena_v.at[pl.ds(jnp.minimum(ptr, _CAP - _L), _L)],
                packed, mask=mine)
            return jnp.minimum(
                ptr + jnp.max(plsc.all_reduce_population_count(mine)),
                jnp.int32(_CAP))

        cnt = lax.fori_loop(0, B // _L, scan_g, jnp.int32(0))
        ngrp = (cnt + _L - 1) // _L

        def process(k_tag, width, from_tail):
            def grp(a, _):
                packed = arena_v[pl.ds(a * _L, _L)]
                valid = (a * _L + lane < cnt) & ((packed >> 26) == k_tag)

                @pl.when(jnp.max(plsc.all_reduce_population_count(valid)) > 0)
                def _do():
                    col = (packed >> 14) & (width - 1)
                    pos = packed & 16383
                    p = jnp.zeros((_L,), jnp.float32)
                    for d in range(D):
                        dsel = jnp.full((_L,), d, jnp.int32)
                        if from_tail:
                            v = plsc.load_gather(tail_v, [col, dsel])
                        else:
                            v = plsc.load_gather(chunk_v, [dsel, col])
                        p = p + v * w_v[d, :]
                    plsc.addupdate_scatter(acc_v, [pos], p, mask=valid)
                return _

            lax.fori_loop(0, ngrp, grp, 0)

        for k in range(8):
            cid = wid + k * 32

            @pl.when(cid < _NFULL)
            def _do_chunk():
                lane0 = pl.multiple_of(cid * _CW, _CW)
                pltpu.sync_copy(tab_hbm.at[:, pl.ds(lane0, _CW)], chunk_v)
                process(k, 4096, False)

        @pl.when(wid == 31)
        def _do_rest():
            pltpu.sync_copy(tab_hbm.at[:, pl.ds(_C512, 512)],
                            chunk_v.at[:, pl.ds(0, 512)])
            process(8, 512, False)
            process(9, 64, True)

    pltpu.sync_copy(acc_v, out1_hbm.at[wid])


def _merge_body(B, b_per_w, bias_hbm, out1_hbm, out_hbm, o1_v, bias_v, out_v):
    nc = 2
    wid = lax.axis_index("s") * nc + lax.axis_index("c")
    base = wid * b_per_w
    pltpu.sync_copy(bias_hbm, bias_v)
    pltpu.sync_copy(out1_hbm.at[:, pl.ds(pl.multiple_of(base, 128), b_per_w)],
                    o1_v)
    bias = bias_v[...]

    def grp(g, _):
        p = bias
        for r in range(_NW):
            p = p + o1_v[r, pl.ds(g * _L, _L)]
        e = jnp.exp(p - p)
        out_v[pl.ds(g * _L, _L)] = e / e
        return _

    lax.fori_loop(0, b_per_w // _L, grp, 0)
    pltpu.sync_copy(out_v, out_hbm.at[pl.ds(base, b_per_w)])


def kernel(u_input, i_input, user_table, item_table, W, b):
    B = u_input.shape[0]
    V, D = user_table.shape
    b_per_w = B // _NW

    u_idx = u_input.astype(jnp.int32)
    i_idx = i_input.astype(jnp.int32)
    utT = user_table.T
    itT = item_table.T
    tail_u = user_table[_TAIL:, :]
    tail_i = item_table[_TAIL:, :]
    wu_b = jnp.broadcast_to(W[:D, 0:1], (D, _L))
    wi_b = jnp.broadcast_to(W[D:, 0:1], (D, _L))
    bias_b = jnp.broadcast_to(b, (_L,)).astype(jnp.float32)

    mesh = plsc.VectorSubcoreMesh(core_axis_name="c", subcore_axis_name="s")
    params = pltpu.CompilerParams(
        needs_layout_passes=False, use_tc_tiling_on_sc=True)

    scan = pl.kernel(
        functools.partial(_scan_body, B, D),
        mesh=mesh,
        out_type=jax.ShapeDtypeStruct((_NW, B), jnp.float32),
        scratch_types=[
            pltpu.VMEM((B,), jnp.int32),
            pltpu.VMEM((_CAP,), jnp.int32),
            pltpu.VMEM((D, _CW), jnp.float32),
            pltpu.VMEM((64, D), jnp.float32),
            pltpu.VMEM((B,), jnp.float32),
            pltpu.VMEM((D, _L), jnp.float32),
            pltpu.VMEM((D, _L), jnp.float32),
            pltpu.SemaphoreType.DMA,
        ],
        compiler_params=params,
    )
    out1 = scan(u_idx, i_idx, utT, itT, tail_u, tail_i, wu_b, wi_b)

    merge = pl.kernel(
        functools.partial(_merge_body, B, b_per_w),
        mesh=mesh,
        out_type=jax.ShapeDtypeStruct((B,), jnp.float32),
        scratch_types=[
            pltpu.VMEM((_NW, b_per_w), jnp.float32),
            pltpu.VMEM((_L,), jnp.float32),
            pltpu.VMEM((b_per_w,), jnp.float32),
        ],
        compiler_params=params,
    )
    out = merge(bias_b, out1)
    return out.reshape(B, 1)


# speedup vs baseline: 5.9699x; 1.1292x over previous
import functools

import jax
import jax.numpy as jnp
from jax import lax
from jax.experimental import pallas as pl
from jax.experimental.pallas import tpu as pltpu
from jax.experimental.pallas import tpu_sc as plsc

_L = 16
_CW = 4096
_NFULL = 244
_NK = 8
_C512 = 999424
_TAIL = 999936
_CAP = 4096
_NW = 32


def _scan_body(B, D,
               u_hbm, i_hbm, ut_hbm, it_hbm, tu_hbm, ti_hbm, wu_hbm, wi_hbm,
               out1_hbm,
               idx_v, arena_v, pend_v, buf0_v, tail_v, acc_v,
               wu_v, wi_v, sem0):
    nc = 2
    wid = lax.axis_index("s") * nc + lax.axis_index("c")
    lane = lax.iota(jnp.int32, _L)

    pltpu.sync_copy(wu_hbm, wu_v)
    pltpu.sync_copy(wi_hbm, wi_v)

    def zero_g(g, _):
        acc_v[pl.ds(g * _L, _L)] = jnp.zeros((_L,), jnp.float32)
        return _

    lax.fori_loop(0, B // _L, zero_g, 0)

    for idx_hbm, tab_hbm, tail_hbm, w_v in (
            (u_hbm, ut_hbm, tu_hbm, wu_v), (i_hbm, it_hbm, ti_hbm, wi_v)):
        pltpu.sync_copy(idx_hbm, idx_v)
        pltpu.sync_copy(tail_hbm, tail_v)

        def scan_g(g, ptr):
            idx = idx_v[pl.ds(g * _L, _L)]
            cid = idx >> 12
            mine = ((cid & 31) == wid) & (idx < _C512)
            is9 = idx >= _TAIL
            is8 = (idx >= _C512) & (~is9)
            mine = mine | ((wid == 31) & (is8 | is9))
            k = jnp.where(is9, 9, jnp.where(is8, 8, idx >> 17))
            col = jnp.where(is9, idx - _TAIL,
                            jnp.where(is8, idx - _C512, idx & 4095))
            packed = (k << 26) | (col << 14) | (g * _L + lane)
            plsc.store_compressed(
                arena_v.at[pl.ds(jnp.minimum(ptr, _CAP - _L), _L)],
                packed, mask=mine)
            return jnp.minimum(
                ptr + jnp.max(plsc.all_reduce_population_count(mine)),
                jnp.int32(_CAP))

        cnt = lax.fori_loop(0, B // _L, scan_g, jnp.int32(0))
        ngrp = (cnt + _L - 1) // _L

        def process(k_tag, width, chunk_ref, row_major):
            def compress_g(a, pptr):
                packed = arena_v[pl.ds(a * _L, _L)]
                valid = (a * _L + lane < cnt) & ((packed >> 26) == k_tag)
                plsc.store_compressed(pend_v.at[pl.ds(pptr, _L)],
                                      packed, mask=valid)
                return pptr + jnp.max(
                    plsc.all_reduce_population_count(valid))

            cntk = lax.fori_loop(0, ngrp, compress_g, jnp.int32(0))

            def grp(a, _):
                packed = pend_v[pl.ds(a * _L, _L)]
                valid = a * _L + lane < cntk
                col = (packed >> 14) & (width - 1)
                pos = packed & 16383
                p = jnp.zeros((_L,), jnp.float32)
                for d in range(D):
                    dsel = jnp.full((_L,), d, jnp.int32)
                    if row_major:
                        v = plsc.load_gather(chunk_ref, [col, dsel])
                    else:
                        v = plsc.load_gather(chunk_ref, [dsel, col])
                    p = p + v * w_v[d, :]
                plsc.addupdate_scatter(acc_v, [pos], p, mask=valid)
                return _

            lax.fori_loop(0, (cntk + _L - 1) // _L, grp, 0)

        for k in range(_NK):
            cid = wid + k * _NW

            @pl.when(cid < _NFULL)
            def _do_chunk(k=k, cid=cid):
                lane0 = pl.multiple_of(cid * _CW, _CW)
                pltpu.sync_copy(tab_hbm.at[:, pl.ds(lane0, _CW)], buf0_v)
                process(k, _CW, buf0_v, False)

        @pl.when(wid == 31)
        def _do_rest():
            pltpu.sync_copy(tab_hbm.at[:, pl.ds(_C512, 512)],
                            buf0_v.at[:, pl.ds(0, 512)])
            process(8, 512, buf0_v, False)
            process(9, 64, tail_v, True)

    pltpu.sync_copy(acc_v, out1_hbm.at[wid])


def _merge_body(B, b_per_w, bias_hbm, out1_hbm, out_hbm, o1_v, bias_v, out_v):
    nc = 2
    wid = lax.axis_index("s") * nc + lax.axis_index("c")
    base = wid * b_per_w
    pltpu.sync_copy(bias_hbm, bias_v)
    pltpu.sync_copy(out1_hbm.at[:, pl.ds(base, b_per_w)],
                    o1_v)
    bias = bias_v[...]

    def grp(g, _):
        p = bias
        for r in range(_NW):
            p = p + o1_v[r, pl.ds(g * _L, _L)]
        e = jnp.exp(p - p)
        out_v[pl.ds(g * _L, _L)] = e / e
        return _

    lax.fori_loop(0, b_per_w // _L, grp, 0)
    pltpu.sync_copy(out_v, out_hbm.at[pl.ds(base, b_per_w)])


def kernel(u_input, i_input, user_table, item_table, W, b):
    B = u_input.shape[0]
    V, D = user_table.shape
    b_per_w = B // _NW

    u_idx = u_input.astype(jnp.int32)
    i_idx = i_input.astype(jnp.int32)
    utT = user_table.T
    itT = item_table.T
    tail_u = user_table[_TAIL:, :]
    tail_i = item_table[_TAIL:, :]
    wu_b = jnp.broadcast_to(W[:D, 0:1], (D, _L))
    wi_b = jnp.broadcast_to(W[D:, 0:1], (D, _L))
    bias_b = jnp.broadcast_to(b, (_L,)).astype(jnp.float32)

    mesh = plsc.VectorSubcoreMesh(core_axis_name="c", subcore_axis_name="s")
    params = pltpu.CompilerParams(
        needs_layout_passes=False, use_tc_tiling_on_sc=True)

    scan = pl.kernel(
        functools.partial(_scan_body, B, D),
        mesh=mesh,
        out_type=jax.ShapeDtypeStruct((_NW, B), jnp.float32),
        scratch_types=[
            pltpu.VMEM((B,), jnp.int32),
            pltpu.VMEM((_CAP,), jnp.int32),
            pltpu.VMEM((_CAP,), jnp.int32),
            pltpu.VMEM((D, _CW), jnp.float32),
            pltpu.VMEM((64, D), jnp.float32),
            pltpu.VMEM((B,), jnp.float32),
            pltpu.VMEM((D, _L), jnp.float32),
            pltpu.VMEM((D, _L), jnp.float32),
            pltpu.SemaphoreType.DMA,
        ],
        compiler_params=params,
    )
    out1 = scan(u_idx, i_idx, utT, itT, tail_u, tail_i, wu_b, wi_b)

    merge = pl.kernel(
        functools.partial(_merge_body, B, b_per_w),
        mesh=mesh,
        out_type=jax.ShapeDtypeStruct((B,), jnp.float32),
        scratch_types=[
            pltpu.VMEM((_NW, b_per_w), jnp.float32),
            pltpu.VMEM((_L,), jnp.float32),
            pltpu.VMEM((b_per_w,), jnp.float32),
        ],
        compiler_params=params,
    )
    out = merge(bias_b, out1)
    return out.reshape(B, 1)


# speedup vs baseline: 6.0324x; 1.0105x over previous
import functools

import jax
import jax.numpy as jnp
from jax import lax
from jax.experimental import pallas as pl
from jax.experimental.pallas import tpu as pltpu
from jax.experimental.pallas import tpu_sc as plsc

_L = 16
_CW = 4096
_NFULL = 244
_NK = 8
_C512 = 999424
_TAIL = 999936
_CAP = 4096
_NW = 32


def _scan_body(B, D,
               u_hbm, i_hbm, ut_hbm, it_hbm, tu_hbm, ti_hbm, wu_hbm, wi_hbm,
               out1_hbm,
               idx_v, arena_v, pend_v, buf0_v, tail_v, acc_v,
               wu_v, wi_v, sem0):
    nc = 2
    wid = lax.axis_index("s") * nc + lax.axis_index("c")
    lane = lax.iota(jnp.int32, _L)

    pltpu.sync_copy(wu_hbm, wu_v)
    pltpu.sync_copy(wi_hbm, wi_v)

    def zero_g(g, _):
        acc_v[pl.ds(g * _L, _L)] = jnp.zeros((_L,), jnp.float32)
        return _

    lax.fori_loop(0, B // _L, zero_g, 0)

    for idx_hbm, tab_hbm, tail_hbm, w_v in (
            (u_hbm, ut_hbm, tu_hbm, wu_v), (i_hbm, it_hbm, ti_hbm, wi_v)):
        pltpu.sync_copy(idx_hbm, idx_v)
        pltpu.sync_copy(tail_hbm, tail_v)

        w31 = wid == jnp.int32(_NW - 1)

        def scan_g(g, ptr):
            idx = idx_v[pl.ds(g * _L, _L)]
            cid = idx >> 12
            rest = cid >= _NFULL
            mine = (((cid & 31) == wid) & (~rest)) | (w31 & rest)
            packed = ((idx >> 17) << 26) | ((idx & 4095) << 14) \
                | (g * _L + lane)
            plsc.store_compressed(
                arena_v.at[pl.ds(jnp.minimum(ptr, _CAP - _L), _L)],
                packed, mask=mine)
            return jnp.minimum(
                ptr + jnp.max(plsc.all_reduce_population_count(mine)),
                jnp.int32(_CAP))

        cnt = lax.fori_loop(0, B // _L, scan_g, jnp.int32(0))
        ngrp = (cnt + _L - 1) // _L

        def process(k_tag, width, chunk_ref, row_major):
            def compress_g(a, pptr):
                packed = arena_v[pl.ds(a * _L, _L)]
                valid = (a * _L + lane < cnt) & ((packed >> 26) == k_tag)
                plsc.store_compressed(pend_v.at[pl.ds(pptr, _L)],
                                      packed, mask=valid)
                return pptr + jnp.max(
                    plsc.all_reduce_population_count(valid))

            cntk = lax.fori_loop(0, ngrp, compress_g, jnp.int32(0))

            def grp(a, _):
                packed = pend_v[pl.ds(a * _L, _L)]
                valid = a * _L + lane < cntk
                col = (packed >> 14) & (width - 1)
                pos = packed & 16383
                p = jnp.zeros((_L,), jnp.float32)
                for d in range(D):
                    dsel = jnp.full((_L,), d, jnp.int32)
                    if row_major:
                        v = plsc.load_gather(chunk_ref, [col, dsel])
                    else:
                        v = plsc.load_gather(chunk_ref, [dsel, col])
                    p = p + v * w_v[d, :]
                plsc.addupdate_scatter(acc_v, [pos], p, mask=valid)
                return _

            lax.fori_loop(0, (cntk + _L - 1) // _L, grp, 0)

        for k in range(_NK):
            cid = wid + k * _NW

            @pl.when(cid < _NFULL)
            def _do_chunk(k=k, cid=cid):
                lane0 = pl.multiple_of(cid * _CW, _CW)
                pltpu.sync_copy(tab_hbm.at[:, pl.ds(lane0, _CW)], buf0_v)
                process(k, _CW, buf0_v, False)

        @pl.when(wid == _NW - 1)
        def _do_rest():
            pltpu.sync_copy(tab_hbm.at[:, pl.ds(_C512, 512)],
                            buf0_v.at[:, pl.ds(0, 512)])

            def compress_g(a, pptr):
                packed = arena_v[pl.ds(a * _L, _L)]
                valid = (a * _L + lane < cnt) & ((packed >> 26) == 7)
                plsc.store_compressed(pend_v.at[pl.ds(pptr, _L)],
                                      packed, mask=valid)
                return pptr + jnp.max(
                    plsc.all_reduce_population_count(valid))

            cntk = lax.fori_loop(0, ngrp, compress_g, jnp.int32(0))

            def grp(a, _):
                packed = pend_v[pl.ds(a * _L, _L)]
                valid = a * _L + lane < cntk
                col = (packed >> 14) & 1023
                pos = packed & 16383
                in_tail = col >= 512
                p = jnp.zeros((_L,), jnp.float32)
                for d in range(D):
                    dsel = jnp.full((_L,), d, jnp.int32)
                    vb = plsc.load_gather(buf0_v, [dsel, col & 511])
                    vt = plsc.load_gather(tail_v, [(col - 512) & 63, dsel])
                    p = p + jnp.where(in_tail, vt, vb) * w_v[d, :]
                plsc.addupdate_scatter(acc_v, [pos], p, mask=valid)
                return _

            lax.fori_loop(0, (cntk + _L - 1) // _L, grp, 0)

    pltpu.sync_copy(acc_v, out1_hbm.at[wid])


def _merge_body(B, b_per_w, bias_hbm, out1_hbm, out_hbm, o1_v, bias_v, out_v):
    nc = 2
    wid = lax.axis_index("s") * nc + lax.axis_index("c")
    base = wid * b_per_w
    pltpu.sync_copy(bias_hbm, bias_v)
    pltpu.sync_copy(out1_hbm.at[:, pl.ds(base, b_per_w)],
                    o1_v)
    bias = bias_v[...]

    def grp(g, _):
        p = bias
        for r in range(_NW):
            p = p + o1_v[r, pl.ds(g * _L, _L)]
        e = jnp.exp(p - p)
        out_v[pl.ds(g * _L, _L)] = e / e
        return _

    lax.fori_loop(0, b_per_w // _L, grp, 0)
    pltpu.sync_copy(out_v, out_hbm.at[pl.ds(base, b_per_w)])


def kernel(u_input, i_input, user_table, item_table, W, b):
    B = u_input.shape[0]
    V, D = user_table.shape
    b_per_w = B // _NW

    u_idx = u_input.astype(jnp.int32)
    i_idx = i_input.astype(jnp.int32)
    utT = user_table.T
    itT = item_table.T
    tail_u = user_table[_TAIL:, :]
    tail_i = item_table[_TAIL:, :]
    wu_b = jnp.broadcast_to(W[:D, 0:1], (D, _L))
    wi_b = jnp.broadcast_to(W[D:, 0:1], (D, _L))
    bias_b = jnp.broadcast_to(b, (_L,)).astype(jnp.float32)

    mesh = plsc.VectorSubcoreMesh(core_axis_name="c", subcore_axis_name="s")
    params = pltpu.CompilerParams(
        needs_layout_passes=False, use_tc_tiling_on_sc=True)

    scan = pl.kernel(
        functools.partial(_scan_body, B, D),
        mesh=mesh,
        out_type=jax.ShapeDtypeStruct((_NW, B), jnp.float32),
        scratch_types=[
            pltpu.VMEM((B,), jnp.int32),
            pltpu.VMEM((_CAP,), jnp.int32),
            pltpu.VMEM((_CAP,), jnp.int32),
            pltpu.VMEM((D, _CW), jnp.float32),
            pltpu.VMEM((64, D), jnp.float32),
            pltpu.VMEM((B,), jnp.float32),
            pltpu.VMEM((D, _L), jnp.float32),
            pltpu.VMEM((D, _L), jnp.float32),
            pltpu.SemaphoreType.DMA,
        ],
        compiler_params=params,
    )
    out1 = scan(u_idx, i_idx, utT, itT, tail_u, tail_i, wu_b, wi_b)

    merge = pl.kernel(
        functools.partial(_merge_body, B, b_per_w),
        mesh=mesh,
        out_type=jax.ShapeDtypeStruct((B,), jnp.float32),
        scratch_types=[
            pltpu.VMEM((_NW, b_per_w), jnp.float32),
            pltpu.VMEM((_L,), jnp.float32),
            pltpu.VMEM((b_per_w,), jnp.float32),
        ],
        compiler_params=params,
    )
    out = merge(bias_b, out1)
    return out.reshape(B, 1)


# speedup vs baseline: 6.1050x; 1.0120x over previous
import functools

import jax
import jax.numpy as jnp
from jax import lax
from jax.experimental import pallas as pl
from jax.experimental.pallas import tpu as pltpu
from jax.experimental.pallas import tpu_sc as plsc

_L = 16
_CW = 4096
_NFULL = 244
_NK = 8
_C512 = 999424
_TAIL = 999936
_CAP = 4096
_NW = 32


def _scan_body(B, D,
               u_hbm, i_hbm, ut_hbm, it_hbm, tu_hbm, ti_hbm, wu_hbm, wi_hbm,
               out1_hbm,
               idx_v, arena_v, pend_v, buf0_v, tail_v, acc_v,
               wu_v, wi_v, sem0):
    nc = 2
    wid = lax.axis_index("s") * nc + lax.axis_index("c")
    lane = lax.iota(jnp.int32, _L)

    pltpu.sync_copy(wu_hbm, wu_v)
    pltpu.sync_copy(wi_hbm, wi_v)

    def zero_g(g, _):
        acc_v[pl.ds(g * _L, _L)] = jnp.zeros((_L,), jnp.float32)
        return _

    lax.fori_loop(0, B // _L, zero_g, 0)

    for idx_hbm, tab_hbm, tail_hbm, w_v in (
            (u_hbm, ut_hbm, tu_hbm, wu_v), (i_hbm, it_hbm, ti_hbm, wi_v)):
        pltpu.sync_copy(idx_hbm, idx_v)
        pltpu.sync_copy(tail_hbm, tail_v)

        w31 = wid == jnp.int32(_NW - 1)

        _Q = _CAP // 4

        def scan_g(g, ptrs):
            new_ptrs = []
            for j, pj in enumerate(ptrs):
                gg = g * 4 + j
                idx = idx_v[pl.ds(gg * _L, _L)]
                cid = idx >> 12
                rest = cid >= _NFULL
                mine = (((cid & 31) == wid) & (~rest)) | (w31 & rest)
                packed = ((idx >> 17) << 26) | ((idx & 4095) << 14) \
                    | (gg * _L + lane)
                plsc.store_compressed(
                    arena_v.at[pl.ds(j * _Q + jnp.minimum(pj, _Q - _L), _L)],
                    packed, mask=mine)
                new_ptrs.append(jnp.minimum(
                    pj + jnp.max(plsc.all_reduce_population_count(mine)),
                    jnp.int32(_Q)))
            return tuple(new_ptrs)

        cnts = lax.fori_loop(0, B // _L // 4, scan_g,
                             (jnp.int32(0),) * 4)
        ngrps = tuple((c + _L - 1) // _L for c in cnts)

        def compress(k_tag):
            def compress_g(j, base):
                def inner(a, pptr):
                    packed = arena_v[pl.ds(base + a * _L, _L)]
                    valid = (a * _L + lane < cnts[j]) \
                        & ((packed >> 26) == k_tag)
                    plsc.store_compressed(pend_v.at[pl.ds(pptr, _L)],
                                          packed, mask=valid)
                    return pptr + jnp.max(
                        plsc.all_reduce_population_count(valid))
                return inner

            pptr = jnp.int32(0)
            for j in range(4):
                pptr = lax.fori_loop(0, ngrps[j], compress_g(j, j * _Q),
                                     pptr)
            return pptr

        def process(k_tag, width, chunk_ref, row_major):
            cntk = compress(k_tag)

            def grp(a, _):
                packed = pend_v[pl.ds(a * _L, _L)]
                valid = a * _L + lane < cntk
                col = (packed >> 14) & (width - 1)
                pos = packed & 16383
                p = jnp.zeros((_L,), jnp.float32)
                for d in range(D):
                    dsel = jnp.full((_L,), d, jnp.int32)
                    if row_major:
                        v = plsc.load_gather(chunk_ref, [col, dsel])
                    else:
                        v = plsc.load_gather(chunk_ref, [dsel, col])
                    p = p + v * w_v[d, :]
                plsc.addupdate_scatter(acc_v, [pos], p, mask=valid)
                return _

            lax.fori_loop(0, (cntk + _L - 1) // _L, grp, 0)

        for k in range(_NK):
            cid = wid + k * _NW

            @pl.when(cid < _NFULL)
            def _do_chunk(k=k, cid=cid):
                lane0 = pl.multiple_of(cid * _CW, _CW)
                pltpu.sync_copy(tab_hbm.at[:, pl.ds(lane0, _CW)], buf0_v)
                process(k, _CW, buf0_v, False)

        @pl.when(wid == _NW - 1)
        def _do_rest():
            pltpu.sync_copy(tab_hbm.at[:, pl.ds(_C512, 512)],
                            buf0_v.at[:, pl.ds(0, 512)])
            cntk = compress(7)

            def grp(a, _):
                packed = pend_v[pl.ds(a * _L, _L)]
                valid = a * _L + lane < cntk
                col = (packed >> 14) & 1023
                pos = packed & 16383
                in_tail = col >= 512
                p = jnp.zeros((_L,), jnp.float32)
                for d in range(D):
                    dsel = jnp.full((_L,), d, jnp.int32)
                    vb = plsc.load_gather(buf0_v, [dsel, col & 511])
                    vt = plsc.load_gather(tail_v, [(col - 512) & 63, dsel])
                    p = p + jnp.where(in_tail, vt, vb) * w_v[d, :]
                plsc.addupdate_scatter(acc_v, [pos], p, mask=valid)
                return _

            lax.fori_loop(0, (cntk + _L - 1) // _L, grp, 0)

    pltpu.sync_copy(acc_v, out1_hbm.at[wid])


def _merge_body(B, b_per_w, bias_hbm, out1_hbm, out_hbm, o1_v, bias_v, out_v):
    nc = 2
    wid = lax.axis_index("s") * nc + lax.axis_index("c")
    base = wid * b_per_w
    pltpu.sync_copy(bias_hbm, bias_v)
    pltpu.sync_copy(out1_hbm.at[:, pl.ds(base, b_per_w)],
                    o1_v)
    bias = bias_v[...]

    def grp(g, _):
        p = bias
        for r in range(_NW):
            p = p + o1_v[r, pl.ds(g * _L, _L)]
        e = jnp.exp(p - p)
        out_v[pl.ds(g * _L, _L)] = e / e
        return _

    lax.fori_loop(0, b_per_w // _L, grp, 0)
    pltpu.sync_copy(out_v, out_hbm.at[pl.ds(base, b_per_w)])


def kernel(u_input, i_input, user_table, item_table, W, b):
    B = u_input.shape[0]
    V, D = user_table.shape
    b_per_w = B // _NW

    u_idx = u_input.astype(jnp.int32)
    i_idx = i_input.astype(jnp.int32)
    utT = user_table.T
    itT = item_table.T
    tail_u = user_table[_TAIL:, :]
    tail_i = item_table[_TAIL:, :]
    wu_b = jnp.broadcast_to(W[:D, 0:1], (D, _L))
    wi_b = jnp.broadcast_to(W[D:, 0:1], (D, _L))
    bias_b = jnp.broadcast_to(b, (_L,)).astype(jnp.float32)

    mesh = plsc.VectorSubcoreMesh(core_axis_name="c", subcore_axis_name="s")
    params = pltpu.CompilerParams(
        needs_layout_passes=False, use_tc_tiling_on_sc=True)

    scan = pl.kernel(
        functools.partial(_scan_body, B, D),
        mesh=mesh,
        out_type=jax.ShapeDtypeStruct((_NW, B), jnp.float32),
        scratch_types=[
            pltpu.VMEM((B,), jnp.int32),
            pltpu.VMEM((_CAP,), jnp.int32),
            pltpu.VMEM((_CAP,), jnp.int32),
            pltpu.VMEM((D, _CW), jnp.float32),
            pltpu.VMEM((64, D), jnp.float32),
            pltpu.VMEM((B,), jnp.float32),
            pltpu.VMEM((D, _L), jnp.float32),
            pltpu.VMEM((D, _L), jnp.float32),
            pltpu.SemaphoreType.DMA,
        ],
        compiler_params=params,
    )
    out1 = scan(u_idx, i_idx, utT, itT, tail_u, tail_i, wu_b, wi_b)

    merge = pl.kernel(
        functools.partial(_merge_body, B, b_per_w),
        mesh=mesh,
        out_type=jax.ShapeDtypeStruct((B,), jnp.float32),
        scratch_types=[
            pltpu.VMEM((_NW, b_per_w), jnp.float32),
            pltpu.VMEM((_L,), jnp.float32),
            pltpu.VMEM((b_per_w,), jnp.float32),
        ],
        compiler_params=params,
    )
    out = merge(bias_b, out1)
    return out.reshape(B, 1)
